# baseline (speedup 1.0000x reference)
import functools

import jax
import jax.numpy as jnp
from jax import lax
from jax.experimental import pallas as pl
from jax.experimental.pallas import tpu as pltpu
from jax.experimental.pallas import tpu_sc as plsc

N = 10000
E = 320000
D = 128

NC = 2
NS = 16
NW = NC * NS
CH = 64
NCH = 160
E2 = NW * NCH * CH
TRASH = N
ACCN = 10008
R_MAIN = 624
R_LAST = ACCN - (NS - 1) * R_MAIN
NP = 10240
DB = NP // 128


def _mm_body(x_ref, wl_ref, wr_ref, xl_ref, xr_ref):
    x = x_ref[...]
    xl_ref[...] = jnp.dot(x, wl_ref[...], preferred_element_type=jnp.float32)
    xr_ref[...] = jnp.dot(x, wr_ref[...], preferred_element_type=jnp.float32)


def _fin_body(acc_ref, den_ref, bias_ref, gamma_ref, beta_ref, out_ref):
    acc = acc_ref[0] + acc_ref[1]
    den2 = den_ref[0] + den_ref[1]
    oh = (lax.broadcasted_iota(jnp.int32, (1, DB), 1) == pl.program_id(0))
    den_col = jax.lax.dot_general(
        den2, oh.astype(jnp.float32), (((0,), (1,)), ((), ())),
        preferred_element_type=jnp.float32)
    out = acc / (den_col + 1e-16) + bias_ref[...]
    mu = jnp.mean(out, axis=-1, keepdims=True)
    var = jnp.mean((out - mu) ** 2, axis=-1, keepdims=True)
    out_ref[...] = (out - mu) / jnp.sqrt(var + 1e-5) * gamma_ref[...] + beta_ref[...]


def _edge_body(xl_hbm, xr_hbm, src_hbm, dst_hbm, att_hbm, z128_hbm, z1d_hbm,
               outp_hbm, denp_hbm,
               att_v, si0, si1, di0, di1,
               rl0, rl1, rr0, rr1, ee0, ee1,
               acc_sp, den_sp,
               gl0, gl1, gr0, gr1, sa0, sa1, sd0, sd1, ix0, ix1):
    SI = (si0, si1)
    DI = (di0, di1)
    RL = (rl0, rl1)
    RR = (rr0, rr1)
    EE = (ee0, ee1)
    GL = (gl0, gl1)
    GR = (gr0, gr1)
    SA = (sa0, sa1)
    SD = (sd0, sd1)
    IX = (ix0, ix1)

    c = lax.axis_index("c")
    s = lax.axis_index("s")
    wid = s * NC + c
    r0 = pl.multiple_of(s * R_MAIN, 8)
    lanes = lax.iota(jnp.int32, 16)

    @pl.when(s < NS - 1)
    def _zero_main():
        pltpu.sync_copy(z128_hbm.at[pl.ds(r0, R_MAIN)],
                        acc_sp.at[pl.ds(r0, R_MAIN)])

    @pl.when(s == NS - 1)
    def _zero_last():
        pltpu.sync_copy(z128_hbm.at[pl.ds((NS - 1) * R_MAIN, R_LAST)],
                        acc_sp.at[pl.ds((NS - 1) * R_MAIN, R_LAST)])

    d0 = pl.multiple_of(s * (NP // NS), 8)
    pltpu.sync_copy(z1d_hbm.at[pl.ds(d0, NP // NS)],
                    den_sp.at[pl.ds(d0, NP // NS)])

    pltpu.sync_copy(att_hbm, att_v)
    plsc.subcore_barrier()

    att_regs = [att_v[pl.ds(16 * j, 16)] for j in range(8)]
    perms = [lanes ^ sh for sh in (1, 2, 4, 8)]
    ebase = wid * (NCH * CH)

    def iissue(j, b3):
        off = pl.multiple_of(ebase + j * CH, 8)
        pltpu.async_copy(src_hbm.at[pl.ds(off, CH)], SI[b3], IX[b3])
        pltpu.async_copy(dst_hbm.at[pl.ds(off, CH)], DI[b3], IX[b3])

    def iwait(j, b3):
        off = pl.multiple_of(ebase + j * CH, 8)
        pltpu.make_async_copy(src_hbm.at[pl.ds(off, CH)], SI[b3], IX[b3]).wait()
        pltpu.make_async_copy(dst_hbm.at[pl.ds(off, CH)], DI[b3], IX[b3]).wait()

    def gissue(b3, b2):
        pltpu.async_copy(xl_hbm.at[SI[b3]], RL[b2], GL[b2])
        pltpu.async_copy(xr_hbm.at[DI[b3]], RR[b2], GR[b2])

    def gwait(b3, b2):
        pltpu.make_async_copy(xl_hbm.at[SI[b3]], RL[b2], GL[b2]).wait()
        pltpu.make_async_copy(xr_hbm.at[DI[b3]], RR[b2], GR[b2]).wait()

    def sissue(b3, b2):
        pltpu.async_copy(RL[b2], acc_sp.at[DI[b3]], SA[b2], add=True)
        pltpu.async_copy(EE[b2], den_sp.at[DI[b3]], SD[b2], add=True)

    def swait(b3, b2):
        pltpu.make_async_copy(RL[b2], acc_sp.at[DI[b3]], SA[b2]).wait()
        pltpu.make_async_copy(EE[b2], den_sp.at[DI[b3]], SD[b2]).wait()

    def compute(b2):
        rows_l, rows_r, ee_ref = RL[b2], RR[b2], EE[b2]

        def group(g, carry):
            gbase = pl.multiple_of(g * 16, 16)
            ee_lane = jnp.zeros((16,), jnp.float32)
            for t in range(16):
                e = gbase + t
                acc = jnp.zeros((16,), jnp.float32)
                ls = []
                for j in range(8):
                    sl = pl.ds(16 * j, 16)
                    l = rows_l[e, sl]
                    ls.append(l)
                    sm = l + rows_r[e, sl]
                    sm = jnp.maximum(sm, sm * 0.2)
                    acc = acc + sm * att_regs[j]
                for p in perms:
                    acc = acc + acc[p]
                ee = jnp.exp(acc)
                for j in range(8):
                    rows_l[e, pl.ds(16 * j, 16)] = ls[j] * ee
                ee_lane = jnp.where(lanes == t, ee, ee_lane)
            ee_ref[pl.ds(gbase, 16)] = ee_lane
            return carry

        lax.fori_loop(0, CH // 16, group, 0)

    def step(j, b2):
        b = 1 - b2

        @pl.when(j + 1 < NCH)
        def _ii():
            iissue(j + 1, b)

        gwait(b2, b2)

        @pl.when(j + 1 < NCH)
        def _gi():
            iwait(j + 1, b)
            gissue(b, b)

        compute(b2)
        pltpu.sync_copy(RL[b2], acc_sp.at[DI[b2]], add=True)
        pltpu.sync_copy(EE[b2], den_sp.at[DI[b2]], add=True)

    iissue(0, 0)
    iwait(0, 0)
    gissue(0, 0)

    def loop(u, carry):
        step(2 * u, 0)
        step(2 * u + 1, 1)
        return carry

    lax.fori_loop(0, NCH // 2, loop, 0)

    plsc.subcore_barrier()

    @pl.when(s < NS - 1)
    def _out_main():
        pltpu.sync_copy(acc_sp.at[pl.ds(r0, R_MAIN)],
                        outp_hbm.at[c, pl.ds(r0, R_MAIN)])

    @pl.when(s == NS - 1)
    def _out_last():
        pltpu.sync_copy(acc_sp.at[pl.ds((NS - 1) * R_MAIN, R_LAST)],
                        outp_hbm.at[c, pl.ds((NS - 1) * R_MAIN, R_LAST)])

    pltpu.sync_copy(den_sp.at[pl.ds(d0, NP // NS)],
                    denp_hbm.at[c, pl.ds(d0, NP // NS)])


_edge_kernel = functools.partial(
    pl.kernel,
    out_type=(jax.ShapeDtypeStruct((NC, ACCN, D), jnp.float32),
              jax.ShapeDtypeStruct((NC, NP), jnp.float32)),
    mesh=plsc.VectorSubcoreMesh(core_axis_name="c", subcore_axis_name="s"),
    scratch_types=[
        pltpu.VMEM((D,), jnp.float32),
        pltpu.VMEM((CH,), jnp.int32),
        pltpu.VMEM((CH,), jnp.int32),
        pltpu.VMEM((CH,), jnp.int32),
        pltpu.VMEM((CH,), jnp.int32),
        pltpu.VMEM((CH, D), jnp.float32),
        pltpu.VMEM((CH, D), jnp.float32),
        pltpu.VMEM((CH, D), jnp.float32),
        pltpu.VMEM((CH, D), jnp.float32),
        pltpu.VMEM((CH,), jnp.float32),
        pltpu.VMEM((CH,), jnp.float32),
        pltpu.VMEM_SHARED((ACCN, D), jnp.float32),
        pltpu.VMEM_SHARED((NP,), jnp.float32),
    ] + [pltpu.SemaphoreType.DMA] * 10,
)(_edge_body)


@jax.jit
def kernel(x, edge_index, W_l, W_r, att, bias, ln_gamma, ln_beta):
    src = edge_index[0].astype(jnp.int32)
    dst = edge_index[1].astype(jnp.int32)
    pad = E2 - E
    src1 = jnp.concatenate([src, jnp.zeros((pad,), jnp.int32)])
    dst1 = jnp.concatenate([dst, jnp.full((pad,), TRASH, jnp.int32)])

    bn = 1000
    xl, xr = pl.pallas_call(
        _mm_body,
        grid=(N // bn,),
        in_specs=[
            pl.BlockSpec((bn, D), lambda i: (i, 0)),
            pl.BlockSpec((D, D), lambda i: (0, 0)),
            pl.BlockSpec((D, D), lambda i: (0, 0)),
        ],
        out_specs=[
            pl.BlockSpec((bn, D), lambda i: (i, 0)),
            pl.BlockSpec((bn, D), lambda i: (i, 0)),
        ],
        out_shape=[
            jax.ShapeDtypeStruct((N, D), jnp.float32),
            jax.ShapeDtypeStruct((N, D), jnp.float32),
        ],
    )(x, W_l, W_r)

    z128 = jnp.zeros((ACCN, D), jnp.float32)
    z1d = jnp.zeros((NP,), jnp.float32)
    outp, denp = _edge_kernel(xl, xr, src1, dst1, att, z128, z1d)
    denp = denp.reshape(NC, DB, 128)

    nblk = pl.cdiv(N, 128)
    out = pl.pallas_call(
        _fin_body,
        grid=(nblk,),
        in_specs=[
            pl.BlockSpec((NC, 128, D), lambda i: (0, i, 0)),
            pl.BlockSpec((NC, DB, 128), lambda i: (0, 0, 0)),
            pl.BlockSpec((1, D), lambda i: (0, 0)),
            pl.BlockSpec((1, D), lambda i: (0, 0)),
            pl.BlockSpec((1, D), lambda i: (0, 0)),
        ],
        out_specs=pl.BlockSpec((128, D), lambda i: (i, 0)),
        out_shape=jax.ShapeDtypeStruct((N, D), jnp.float32),
    )(outp, denp, bias.reshape(1, D), ln_gamma.reshape(1, D),
      ln_beta.reshape(1, D))
    return out


# speedup vs baseline: 1.0692x; 1.0692x over previous
import functools

import jax
import jax.numpy as jnp
from jax import lax
from jax.experimental import pallas as pl
from jax.experimental.pallas import tpu as pltpu
from jax.experimental.pallas import tpu_sc as plsc

N = 10000
E = 320000
D = 128

NC = 2
NS = 16
NW = NC * NS
CH = 48
NCH = 210
E2 = NW * NCH * CH
TRASH = N
ACCN = 10008
R_MAIN = 624
R_LAST = ACCN - (NS - 1) * R_MAIN
NP = 10240
DB = NP // 128


def _mm_body(x_ref, wl_ref, wr_ref, xl_ref, xr_ref):
    x = x_ref[...]
    xl_ref[...] = jnp.dot(x, wl_ref[...], preferred_element_type=jnp.float32)
    xr_ref[...] = jnp.dot(x, wr_ref[...], preferred_element_type=jnp.float32)


def _fin_body(acc_ref, den_ref, bias_ref, gamma_ref, beta_ref, out_ref):
    acc = acc_ref[0] + acc_ref[1]
    den2 = den_ref[0] + den_ref[1]
    oh = (lax.broadcasted_iota(jnp.int32, (1, DB), 1) == pl.program_id(0))
    den_col = jax.lax.dot_general(
        den2, oh.astype(jnp.float32), (((0,), (1,)), ((), ())),
        preferred_element_type=jnp.float32)
    out = acc / (den_col + 1e-16) + bias_ref[...]
    mu = jnp.mean(out, axis=-1, keepdims=True)
    var = jnp.mean((out - mu) ** 2, axis=-1, keepdims=True)
    out_ref[...] = (out - mu) / jnp.sqrt(var + 1e-5) * gamma_ref[...] + beta_ref[...]


def _edge_body(xl_hbm, xr_hbm, src_hbm, dst_hbm, att_hbm, z128_hbm, z1d_hbm,
               outp_hbm, denp_hbm,
               att_v, si0, si1, si2, di0, di1, di2,
               rl0, rl1, rl2, rr0, rr1, ee0, ee1, ee2,
               acc_sp, den_sp,
               gl0, gl1, gl2, gr0, gr1, ix0, ix1, ix2,
               sa0, sa1, sa2, sd0, sd1, sd2):
    SI = (si0, si1, si2)
    DI = (di0, di1, di2)
    RL = (rl0, rl1, rl2)
    RR = (rr0, rr1)
    EE = (ee0, ee1, ee2)
    GL = (gl0, gl1, gl2)
    GR = (gr0, gr1)
    IX = (ix0, ix1, ix2)
    SA = (sa0, sa1, sa2)
    SD = (sd0, sd1, sd2)

    c = lax.axis_index("c")
    s = lax.axis_index("s")
    wid = s * NC + c
    r0 = pl.multiple_of(s * R_MAIN, 8)
    lanes = lax.iota(jnp.int32, 16)

    @pl.when(s < NS - 1)
    def _zero_main():
        pltpu.sync_copy(z128_hbm.at[pl.ds(r0, R_MAIN)],
                        acc_sp.at[pl.ds(r0, R_MAIN)])

    @pl.when(s == NS - 1)
    def _zero_last():
        pltpu.sync_copy(z128_hbm.at[pl.ds((NS - 1) * R_MAIN, R_LAST)],
                        acc_sp.at[pl.ds((NS - 1) * R_MAIN, R_LAST)])

    d0 = pl.multiple_of(s * (NP // NS), 8)
    pltpu.sync_copy(z1d_hbm.at[pl.ds(d0, NP // NS)],
                    den_sp.at[pl.ds(d0, NP // NS)])

    pltpu.sync_copy(att_hbm, att_v)
    plsc.subcore_barrier()

    att_regs = [att_v[pl.ds(16 * j, 16)] for j in range(8)]
    perms = [lanes ^ sh for sh in (1, 2, 4, 8)]
    ebase = wid * (NCH * CH)

    def iissue(j, b3):
        off = pl.multiple_of(ebase + j * CH, 8)
        pltpu.async_copy(src_hbm.at[pl.ds(off, CH)], SI[b3], IX[b3])
        pltpu.async_copy(dst_hbm.at[pl.ds(off, CH)], DI[b3], IX[b3])

    def iwait(j, b3):
        off = pl.multiple_of(ebase + j * CH, 8)
        pltpu.make_async_copy(src_hbm.at[pl.ds(off, CH)], SI[b3], IX[b3]).wait()
        pltpu.make_async_copy(dst_hbm.at[pl.ds(off, CH)], DI[b3], IX[b3]).wait()

    def gissue(b3, b2):
        pltpu.async_copy(xl_hbm.at[SI[b3]], RL[b3], GL[b3])
        pltpu.async_copy(xr_hbm.at[DI[b3]], RR[b2], GR[b2])

    def gwait(b3, b2):
        pltpu.make_async_copy(xl_hbm.at[SI[b3]], RL[b3], GL[b3]).wait()
        pltpu.make_async_copy(xr_hbm.at[DI[b3]], RR[b2], GR[b2]).wait()

    def sissue(b3):
        pltpu.async_copy(RL[b3], acc_sp.at[DI[b3]], SA[b3], add=True)
        pltpu.async_copy(EE[b3], den_sp.at[DI[b3]], SD[b3], add=True)

    def swait(b3):
        pltpu.make_async_copy(RL[b3], acc_sp.at[DI[b3]], SA[b3]).wait()
        pltpu.make_async_copy(EE[b3], den_sp.at[DI[b3]], SD[b3]).wait()

    def compute(b3, b2):
        rows_l, rows_r, ee_ref = RL[b3], RR[b2], EE[b3]

        def group(g, carry):
            gbase = pl.multiple_of(g * 16, 16)
            ee_lane = jnp.zeros((16,), jnp.float32)
            for t in range(16):
                e = gbase + t
                acc = jnp.zeros((16,), jnp.float32)
                ls = []
                for j in range(8):
                    sl = pl.ds(16 * j, 16)
                    l = rows_l[e, sl]
                    ls.append(l)
                    sm = l + rows_r[e, sl]
                    sm = jnp.maximum(sm, sm * 0.2)
                    acc = acc + sm * att_regs[j]
                for p in perms:
                    acc = acc + acc[p]
                ee = jnp.exp(acc)
                for j in range(8):
                    rows_l[e, pl.ds(16 * j, 16)] = ls[j] * ee
                ee_lane = jnp.where(lanes == t, ee, ee_lane)
            ee_ref[pl.ds(gbase, 16)] = ee_lane
            return carry

        lax.fori_loop(0, CH // 16, group, 0)

    def step(j, b3, b2):
        @pl.when(j >= 2)
        def _sw():
            swait((b3 + 1) % 3)

        @pl.when(j + 1 < NCH)
        def _ii():
            iissue(j + 1, (b3 + 1) % 3)

        gwait(b3, b2)
        compute(b3, b2)

        @pl.when(j + 1 < NCH)
        def _gi():
            iwait(j + 1, (b3 + 1) % 3)
            gissue((b3 + 1) % 3, 1 - b2)

        sissue(b3)

    iissue(0, 0)
    iwait(0, 0)
    gissue(0, 0)

    def loop(u, carry):
        j0 = 6 * u
        step(j0 + 0, 0, 0)
        step(j0 + 1, 1, 1)
        step(j0 + 2, 2, 0)
        step(j0 + 3, 0, 1)
        step(j0 + 4, 1, 0)
        step(j0 + 5, 2, 1)
        return carry

    lax.fori_loop(0, NCH // 6, loop, 0)
    swait((NCH - 2) % 3)
    swait((NCH - 1) % 3)

    plsc.subcore_barrier()

    @pl.when(s < NS - 1)
    def _out_main():
        pltpu.sync_copy(acc_sp.at[pl.ds(r0, R_MAIN)],
                        outp_hbm.at[c, pl.ds(r0, R_MAIN)])

    @pl.when(s == NS - 1)
    def _out_last():
        pltpu.sync_copy(acc_sp.at[pl.ds((NS - 1) * R_MAIN, R_LAST)],
                        outp_hbm.at[c, pl.ds((NS - 1) * R_MAIN, R_LAST)])

    pltpu.sync_copy(den_sp.at[pl.ds(d0, NP // NS)],
                    denp_hbm.at[c, pl.ds(d0, NP // NS)])


_edge_kernel = functools.partial(
    pl.kernel,
    out_type=(jax.ShapeDtypeStruct((NC, ACCN, D), jnp.float32),
              jax.ShapeDtypeStruct((NC, NP), jnp.float32)),
    mesh=plsc.VectorSubcoreMesh(core_axis_name="c", subcore_axis_name="s"),
    scratch_types=[
        pltpu.VMEM((D,), jnp.float32),
        pltpu.VMEM((CH,), jnp.int32),
        pltpu.VMEM((CH,), jnp.int32),
        pltpu.VMEM((CH,), jnp.int32),
        pltpu.VMEM((CH,), jnp.int32),
        pltpu.VMEM((CH,), jnp.int32),
        pltpu.VMEM((CH,), jnp.int32),
        pltpu.VMEM((CH, D), jnp.float32),
        pltpu.VMEM((CH, D), jnp.float32),
        pltpu.VMEM((CH, D), jnp.float32),
        pltpu.VMEM((CH, D), jnp.float32),
        pltpu.VMEM((CH, D), jnp.float32),
        pltpu.VMEM((CH,), jnp.float32),
        pltpu.VMEM((CH,), jnp.float32),
        pltpu.VMEM((CH,), jnp.float32),
        pltpu.VMEM_SHARED((ACCN, D), jnp.float32),
        pltpu.VMEM_SHARED((NP,), jnp.float32),
    ] + [pltpu.SemaphoreType.DMA] * 14,
)(_edge_body)


@jax.jit
def kernel(x, edge_index, W_l, W_r, att, bias, ln_gamma, ln_beta):
    src = edge_index[0].astype(jnp.int32)
    dst = edge_index[1].astype(jnp.int32)
    pad = E2 - E
    src1 = jnp.concatenate([src, jnp.zeros((pad,), jnp.int32)])
    dst1 = jnp.concatenate([dst, jnp.full((pad,), TRASH, jnp.int32)])

    bn = 1000
    xl, xr = pl.pallas_call(
        _mm_body,
        grid=(N // bn,),
        in_specs=[
            pl.BlockSpec((bn, D), lambda i: (i, 0)),
            pl.BlockSpec((D, D), lambda i: (0, 0)),
            pl.BlockSpec((D, D), lambda i: (0, 0)),
        ],
        out_specs=[
            pl.BlockSpec((bn, D), lambda i: (i, 0)),
            pl.BlockSpec((bn, D), lambda i: (i, 0)),
        ],
        out_shape=[
            jax.ShapeDtypeStruct((N, D), jnp.float32),
            jax.ShapeDtypeStruct((N, D), jnp.float32),
        ],
    )(x, W_l, W_r)

    z128 = jnp.zeros((ACCN, D), jnp.float32)
    z1d = jnp.zeros((NP,), jnp.float32)
    outp, denp = _edge_kernel(xl, xr, src1, dst1, att, z128, z1d)
    denp = denp.reshape(NC, DB, 128)

    nblk = pl.cdiv(N, 128)
    out = pl.pallas_call(
        _fin_body,
        grid=(nblk,),
        in_specs=[
            pl.BlockSpec((NC, 128, D), lambda i: (0, i, 0)),
            pl.BlockSpec((NC, DB, 128), lambda i: (0, 0, 0)),
            pl.BlockSpec((1, D), lambda i: (0, 0)),
            pl.BlockSpec((1, D), lambda i: (0, 0)),
            pl.BlockSpec((1, D), lambda i: (0, 0)),
        ],
        out_specs=pl.BlockSpec((128, D), lambda i: (i, 0)),
        out_shape=jax.ShapeDtypeStruct((N, D), jnp.float32),
    )(outp, denp, bias.reshape(1, D), ln_gamma.reshape(1, D),
      ln_beta.reshape(1, D))
    return out
